# Initial kernel scaffold; baseline (speedup 1.0000x reference)
import jax
import jax.numpy as jnp
from jax.experimental import pallas as pl


def kernel(token_ids, weight):
    raise NotImplementedError("write your pallas kernel here")


# speedup vs baseline: 1.5149x; 1.5149x over previous
import functools

import jax
import jax.numpy as jnp
from jax import lax
from jax.experimental import pallas as pl
from jax.experimental.pallas import tpu as pltpu
from jax.experimental.pallas import tpu_sc as plsc

_NUM_EMB = 1000000
_D = 32
_BATCH = 16384
_HIST = 20
_TOTAL = _BATCH * _HIST

_INFO = plsc.get_sparse_core_info()
_NC = _INFO.num_cores
_NS = _INFO.num_subcores
_NW = _NC * _NS
_B_PER_W = _TOTAL // _NW
_CHUNK = 1024
_N_CHUNKS = _B_PER_W // _CHUNK

_mesh = plsc.VectorSubcoreMesh(core_axis_name="c", subcore_axis_name="s")


@functools.partial(
    pl.kernel,
    mesh=_mesh,
    compiler_params=pltpu.CompilerParams(use_tc_tiling_on_sc=False),
    out_type=jax.ShapeDtypeStruct((_TOTAL, _D), jnp.float32),
    scratch_types=[
        pltpu.VMEM((_B_PER_W,), jnp.int32),
        pltpu.VMEM((2, _CHUNK, _D), jnp.float32),
        pltpu.SemaphoreType.DMA,
        pltpu.SemaphoreType.DMA,
    ],
)
def _gather_kernel(idx_hbm, table_hbm, out_hbm, idx_v, rows_v, gsem, osem):
    wid = lax.axis_index("s") * _NC + lax.axis_index("c")
    base = wid * _B_PER_W
    pltpu.sync_copy(idx_hbm.at[pl.ds(base, _B_PER_W)], idx_v)

    copies = [None, None]
    pltpu.async_copy(
        table_hbm.at[idx_v.at[pl.ds(0, _CHUNK)]], rows_v.at[0], gsem
    )
    for c in range(_N_CHUNKS):
        buf = c % 2
        if c + 1 < _N_CHUNKS:
            if copies[1 - buf] is not None:
                copies[1 - buf].wait()
            pltpu.async_copy(
                table_hbm.at[idx_v.at[pl.ds((c + 1) * _CHUNK, _CHUNK)]],
                rows_v.at[1 - buf],
                gsem,
            )
        pltpu.make_async_copy(
            table_hbm.at[idx_v.at[pl.ds(c * _CHUNK, _CHUNK)]],
            rows_v.at[buf],
            gsem,
        ).wait()
        copies[buf] = pltpu.async_copy(
            rows_v.at[buf],
            out_hbm.at[pl.ds(base + c * _CHUNK, _CHUNK)],
            osem,
        )
    copies[0].wait()
    copies[1].wait()


def kernel(token_ids, weight):
    ids = token_ids.reshape(_TOTAL).astype(jnp.int32)
    out = _gather_kernel(ids, weight)
    return out.reshape(_BATCH, _HIST, _D)


# speedup vs baseline: 1.5185x; 1.0023x over previous
import functools

import jax
import jax.numpy as jnp
from jax import lax
from jax.experimental import pallas as pl
from jax.experimental.pallas import tpu as pltpu
from jax.experimental.pallas import tpu_sc as plsc

_NUM_EMB = 1000000
_D = 32
_BATCH = 16384
_HIST = 20

_INFO = plsc.get_sparse_core_info()
_NC = _INFO.num_cores
_NS = _INFO.num_subcores
_NW = _NC * _NS
_ROWS_PER_W = _BATCH // _NW

_mesh = plsc.VectorSubcoreMesh(core_axis_name="c", subcore_axis_name="s")


@functools.partial(
    pl.kernel,
    mesh=_mesh,
    compiler_params=pltpu.CompilerParams(use_tc_tiling_on_sc=False),
    out_type=jax.ShapeDtypeStruct((_BATCH, _HIST, _D), jnp.float32),
    scratch_types=[
        pltpu.VMEM((_HIST, _ROWS_PER_W), jnp.int32),
        pltpu.VMEM((2, _ROWS_PER_W, _D), jnp.float32),
        pltpu.SemaphoreType.DMA,
        pltpu.SemaphoreType.DMA,
    ],
)
def _gather_kernel(idxt_hbm, table_hbm, out_hbm, idx_v, rows_v, gsem, osem):
    wid = lax.axis_index("s") * _NC + lax.axis_index("c")
    base = wid * _ROWS_PER_W
    pltpu.sync_copy(idxt_hbm.at[:, pl.ds(base, _ROWS_PER_W)], idx_v)

    copies = [None, None]
    pltpu.async_copy(table_hbm.at[idx_v.at[0]], rows_v.at[0], gsem)
    for h in range(_HIST):
        buf = h % 2
        if h + 1 < _HIST:
            if copies[1 - buf] is not None:
                copies[1 - buf].wait()
            pltpu.async_copy(
                table_hbm.at[idx_v.at[h + 1]], rows_v.at[1 - buf], gsem
            )
        pltpu.make_async_copy(
            table_hbm.at[idx_v.at[h]], rows_v.at[buf], gsem
        ).wait()
        copies[buf] = pltpu.async_copy(
            rows_v.at[buf],
            out_hbm.at[pl.ds(base, _ROWS_PER_W), h],
            osem,
        )
    copies[0].wait()
    copies[1].wait()


def kernel(token_ids, weight):
    return _gather_kernel(token_ids.T.astype(jnp.int32), weight)


# speedup vs baseline: 2.2644x; 1.4912x over previous
import functools

import jax
import jax.numpy as jnp
from jax import lax
from jax.experimental import pallas as pl
from jax.experimental.pallas import tpu as pltpu
from jax.experimental.pallas import tpu_sc as plsc

_NUM_EMB = 1000000
_D = 32
_BATCH = 16384
_HIST = 20

_INFO = plsc.get_sparse_core_info()
_NC = _INFO.num_cores
_NS = _INFO.num_subcores
_NW = _NC * _NS
_ROWS_PER_W = _BATCH // _NW

_mesh = plsc.VectorSubcoreMesh(core_axis_name="c", subcore_axis_name="s")

_TBLOCK = 8192
_TGROUP = _TBLOCK // 4
_TGRID = -(-_NUM_EMB // _TBLOCK)
_TROWS = _TGRID * _TBLOCK


def _format_body(wt_ref, out_ref):
    x = wt_ref[...]
    parts = [x[:, j * _TGROUP:(j + 1) * _TGROUP].T for j in range(4)]
    out_ref[...] = jnp.concatenate(parts, axis=1)


_format_table = pl.pallas_call(
    _format_body,
    grid=(_TGRID,),
    in_specs=[pl.BlockSpec((_D, _TBLOCK), lambda i: (0, i))],
    out_specs=pl.BlockSpec((_TGROUP, 4 * _D), lambda i: (i, 0)),
    out_shape=jax.ShapeDtypeStruct((_TROWS // 4, 4 * _D), jnp.float32),
)


def _compacted_row(ids):
    i, rem = ids // _TBLOCK, ids % _TBLOCK
    return _TBLOCK * i + 4 * (rem % _TGROUP) + rem // _TGROUP


@functools.partial(
    pl.kernel,
    mesh=_mesh,
    compiler_params=pltpu.CompilerParams(use_tc_tiling_on_sc=False),
    out_type=jax.ShapeDtypeStruct((_BATCH, _HIST, _D), jnp.float32),
    scratch_types=[
        pltpu.VMEM((_HIST, _ROWS_PER_W), jnp.int32),
        pltpu.VMEM((2, _ROWS_PER_W, _D), jnp.float32),
        pltpu.SemaphoreType.DMA,
        pltpu.SemaphoreType.DMA,
    ],
)
def _gather_kernel(idxt_hbm, table_hbm, out_hbm, idx_v, rows_v, gsem, osem):
    wid = lax.axis_index("s") * _NC + lax.axis_index("c")
    base = wid * _ROWS_PER_W
    pltpu.sync_copy(idxt_hbm.at[:, pl.ds(base, _ROWS_PER_W)], idx_v)

    copies = [None, None]
    pltpu.async_copy(table_hbm.at[idx_v.at[0]], rows_v.at[0], gsem)
    for h in range(_HIST):
        buf = h % 2
        if h + 1 < _HIST:
            if copies[1 - buf] is not None:
                copies[1 - buf].wait()
            pltpu.async_copy(
                table_hbm.at[idx_v.at[h + 1]], rows_v.at[1 - buf], gsem
            )
        pltpu.make_async_copy(
            table_hbm.at[idx_v.at[h]], rows_v.at[buf], gsem
        ).wait()
        copies[buf] = pltpu.async_copy(
            rows_v.at[buf],
            out_hbm.at[pl.ds(base, _ROWS_PER_W), h],
            osem,
        )
    copies[0].wait()
    copies[1].wait()


def kernel(token_ids, weight):
    table = _format_table(weight.T).reshape(_TROWS, _D)
    ids = _compacted_row(token_ids.T.astype(jnp.int32))
    return _gather_kernel(ids, table)


# speedup vs baseline: 2.2843x; 1.0088x over previous
import functools

import jax
import jax.numpy as jnp
from jax import lax
from jax.experimental import pallas as pl
from jax.experimental.pallas import tpu as pltpu
from jax.experimental.pallas import tpu_sc as plsc

_NUM_EMB = 1000000
_D = 32
_BATCH = 16384
_HIST = 20

_INFO = plsc.get_sparse_core_info()
_NC = _INFO.num_cores
_NS = _INFO.num_subcores
_NW = _NC * _NS
_ROWS_PER_W = _BATCH // _NW

_mesh = plsc.VectorSubcoreMesh(core_axis_name="c", subcore_axis_name="s")

_TBLOCK = 32768
_TGROUP = _TBLOCK // 4
_TGRID = -(-_NUM_EMB // _TBLOCK)
_TROWS = _TGRID * _TBLOCK


def _format_body(wt_ref, out_ref):
    x = wt_ref[...]
    parts = [x[:, j * _TGROUP:(j + 1) * _TGROUP].T for j in range(4)]
    out_ref[...] = jnp.concatenate(parts, axis=1)


_format_table = pl.pallas_call(
    _format_body,
    grid=(_TGRID,),
    in_specs=[pl.BlockSpec((_D, _TBLOCK), lambda i: (0, i))],
    out_specs=pl.BlockSpec((_TGROUP, 4 * _D), lambda i: (i, 0)),
    out_shape=jax.ShapeDtypeStruct((_TROWS // 4, 4 * _D), jnp.float32),
)


def _compacted_row(ids):
    i, rem = ids // _TBLOCK, ids % _TBLOCK
    return _TBLOCK * i + 4 * (rem % _TGROUP) + rem // _TGROUP


@functools.partial(
    pl.kernel,
    mesh=_mesh,
    compiler_params=pltpu.CompilerParams(use_tc_tiling_on_sc=False),
    out_type=jax.ShapeDtypeStruct((_BATCH, _HIST, _D), jnp.float32),
    scratch_types=[
        pltpu.VMEM((_HIST, _ROWS_PER_W), jnp.int32),
        pltpu.VMEM((2, _ROWS_PER_W, _D), jnp.float32),
        pltpu.SemaphoreType.DMA,
        pltpu.SemaphoreType.DMA,
    ],
)
def _gather_kernel(idxt_hbm, table_hbm, out_hbm, idx_v, rows_v, gsem, osem):
    wid = lax.axis_index("s") * _NC + lax.axis_index("c")
    base = wid * _ROWS_PER_W
    pltpu.sync_copy(idxt_hbm.at[:, pl.ds(base, _ROWS_PER_W)], idx_v)

    copies = [None, None]
    pltpu.async_copy(table_hbm.at[idx_v.at[0]], rows_v.at[0], gsem)
    for h in range(_HIST):
        buf = h % 2
        if h + 1 < _HIST:
            if copies[1 - buf] is not None:
                copies[1 - buf].wait()
            pltpu.async_copy(
                table_hbm.at[idx_v.at[h + 1]], rows_v.at[1 - buf], gsem
            )
        pltpu.make_async_copy(
            table_hbm.at[idx_v.at[h]], rows_v.at[buf], gsem
        ).wait()
        copies[buf] = pltpu.async_copy(
            rows_v.at[buf],
            out_hbm.at[pl.ds(base, _ROWS_PER_W), h],
            osem,
        )
    copies[0].wait()
    copies[1].wait()


def kernel(token_ids, weight):
    table = _format_table(weight.T).reshape(_TROWS, _D)
    ids = _compacted_row(token_ids.T.astype(jnp.int32))
    return _gather_kernel(ids, table)


# speedup vs baseline: 2.2847x; 1.0002x over previous
import functools

import jax
import jax.numpy as jnp
from jax import lax
from jax.experimental import pallas as pl
from jax.experimental.pallas import tpu as pltpu
from jax.experimental.pallas import tpu_sc as plsc

_NUM_EMB = 1000000
_D = 32
_BATCH = 16384
_HIST = 20

_INFO = plsc.get_sparse_core_info()
_NC = _INFO.num_cores
_NS = _INFO.num_subcores
_NW = _NC * _NS
_ROWS_PER_W = _BATCH // _NW

_mesh = plsc.VectorSubcoreMesh(core_axis_name="c", subcore_axis_name="s")

_TBLOCK = 32768
_TGROUP = _TBLOCK // 4
_TGRID = -(-_NUM_EMB // _TBLOCK)
_TROWS = _TGRID * _TBLOCK


def _format_body(wt_ref, out_ref):
    x = wt_ref[...]
    for j in range(4):
        out_ref[:, j * _D:(j + 1) * _D] = x[:, j * _TGROUP:(j + 1) * _TGROUP].T


_format_table = pl.pallas_call(
    _format_body,
    grid=(_TGRID,),
    in_specs=[pl.BlockSpec((_D, _TBLOCK), lambda i: (0, i))],
    out_specs=pl.BlockSpec((_TGROUP, 4 * _D), lambda i: (i, 0)),
    out_shape=jax.ShapeDtypeStruct((_TROWS // 4, 4 * _D), jnp.float32),
)


def _compacted_row(ids):
    i, rem = ids // _TBLOCK, ids % _TBLOCK
    return _TBLOCK * i + 4 * (rem % _TGROUP) + rem // _TGROUP


@functools.partial(
    pl.kernel,
    mesh=_mesh,
    compiler_params=pltpu.CompilerParams(use_tc_tiling_on_sc=False),
    out_type=jax.ShapeDtypeStruct((_BATCH, _HIST, _D), jnp.float32),
    scratch_types=[
        pltpu.VMEM((_HIST, _ROWS_PER_W), jnp.int32),
        pltpu.VMEM((2, _ROWS_PER_W, _D), jnp.float32),
        pltpu.SemaphoreType.DMA,
        pltpu.SemaphoreType.DMA,
    ],
)
def _gather_kernel(idxt_hbm, table_hbm, out_hbm, idx_v, rows_v, gsem, osem):
    wid = lax.axis_index("s") * _NC + lax.axis_index("c")
    base = wid * _ROWS_PER_W
    pltpu.sync_copy(idxt_hbm.at[:, pl.ds(base, _ROWS_PER_W)], idx_v)

    copies = [None, None]
    pltpu.async_copy(table_hbm.at[idx_v.at[0]], rows_v.at[0], gsem)
    for h in range(_HIST):
        buf = h % 2
        if h + 1 < _HIST:
            if copies[1 - buf] is not None:
                copies[1 - buf].wait()
            pltpu.async_copy(
                table_hbm.at[idx_v.at[h + 1]], rows_v.at[1 - buf], gsem
            )
        pltpu.make_async_copy(
            table_hbm.at[idx_v.at[h]], rows_v.at[buf], gsem
        ).wait()
        copies[buf] = pltpu.async_copy(
            rows_v.at[buf],
            out_hbm.at[pl.ds(base, _ROWS_PER_W), h],
            osem,
        )
    copies[0].wait()
    copies[1].wait()


def kernel(token_ids, weight):
    table = _format_table(weight.T).reshape(_TROWS, _D)
    ids = _compacted_row(token_ids.T.astype(jnp.int32))
    return _gather_kernel(ids, table)
